# Initial kernel scaffold; baseline (speedup 1.0000x reference)
import jax
import jax.numpy as jnp
from jax.experimental import pallas as pl


def kernel(out, x_0, x_1, x_2):
    raise NotImplementedError("write your pallas kernel here")


# speedup vs baseline: 2.0665x; 2.0665x over previous
import jax
import jax.numpy as jnp
import numpy as np
from jax import lax
from jax.experimental import pallas as pl
from jax.experimental.pallas import tpu as pltpu

_OFFS = (0, 7, 15)
_BR = 512
_BC = 512
_G = _BC // 32


def _spread_matrix() -> jax.Array:
    s = np.zeros((3 * _G, _BC), np.float32)
    for t, off in enumerate(_OFFS):
        for g in range(_G):
            s[t * _G + g, 32 * g + off] = 1.0
    return jnp.asarray(s)


def _body(x0_ref, x1_ref, x2_ref, s_ref, in_ref, o_ref):
    j = pl.program_id(1)

    @pl.when(j < 8)
    def _add():
        xcat = jnp.concatenate([x0_ref[...], x1_ref[...], x2_ref[...]], axis=0)
        upd = lax.dot_general(
            xcat,
            s_ref[...],
            (((0,), (0,)), ((), ())),
            preferred_element_type=jnp.float32,
        )
        o_ref[...] = in_ref[...] + upd

    @pl.when(j >= 8)
    def _copy():
        o_ref[...] = in_ref[...]


def kernel(out, x_0, x_1, x_2):
    m, n = out.shape
    grid = (m // _BR, n // _BC)
    s = _spread_matrix()

    def x_map(i, j):
        return jnp.minimum(j, 7), i

    return pl.pallas_call(
        _body,
        grid=grid,
        in_specs=[
            pl.BlockSpec((_G, _BR), x_map),
            pl.BlockSpec((_G, _BR), x_map),
            pl.BlockSpec((_G, _BR), x_map),
            pl.BlockSpec((3 * _G, _BC), lambda i, j: (0, 0)),
            pl.BlockSpec((_BR, _BC), lambda i, j: (i, j)),
        ],
        out_specs=pl.BlockSpec((_BR, _BC), lambda i, j: (i, j)),
        out_shape=jax.ShapeDtypeStruct((m, n), out.dtype),
    )(x_0.T, x_1.T, x_2.T, s, out)
